# baseline (speedup 1.0000x reference)
import functools

import jax
import jax.numpy as jnp
from jax import lax
from jax.experimental import pallas as pl
from jax.experimental.pallas import tpu as pltpu
from jax.experimental.pallas import tpu_sc as plsc

_LANES = 16
_NUM_WORKERS = 32
_CHUNK = 16
_DEPTH = 3


def _sc_embed_call(n_rows, hidden):
    per_w = n_rows // _NUM_WORKERS
    n_chunks = per_w // _CHUNK
    mesh = plsc.VectorSubcoreMesh(core_axis_name="c", subcore_axis_name="s")

    row_bufs = [pltpu.VMEM((_CHUNK, hidden), jnp.float32)
                for _ in range(2 * _DEPTH)]
    sems = [pltpu.SemaphoreType.DMA for _ in range(3 * _DEPTH)]

    @functools.partial(
        pl.kernel,
        mesh=mesh,
        out_type=jax.ShapeDtypeStruct((n_rows, hidden), jnp.float32),
        scratch_types=[
            pltpu.VMEM((per_w,), jnp.int32),
            pltpu.VMEM((per_w,), jnp.int32),
        ] + row_bufs + sems,
    )
    def sc_embed(tok_hbm, pos_hbm, ttab_hbm, ptab_hbm, out_hbm,
                 tidx_v, pidx_v, *scratch):
        wid = lax.axis_index("s") * 2 + lax.axis_index("c")
        base = wid * per_w
        pltpu.sync_copy(tok_hbm.at[pl.ds(base, per_w)], tidx_v)
        pltpu.sync_copy(pos_hbm.at[pl.ds(base, per_w)], pidx_v)

        n_vec = hidden // _LANES
        bufs = [(scratch[2 * b], scratch[2 * b + 1],
                 scratch[2 * _DEPTH + 3 * b],
                 scratch[2 * _DEPTH + 3 * b + 1],
                 scratch[2 * _DEPTH + 3 * b + 2]) for b in range(_DEPTH)]
        gathers = [None] * _DEPTH
        stores = [None] * _DEPTH

        def start_gather(ci, b):
            off = ci * _CHUNK
            tb, pb, s_tg, s_pg, _ = bufs[b]
            if stores[b] is not None:
                stores[b].wait()
                stores[b] = None
            g_t = pltpu.async_copy(
                ttab_hbm.at[tidx_v.at[pl.ds(off, _CHUNK)]], tb, s_tg)
            g_p = pltpu.async_copy(
                ptab_hbm.at[pidx_v.at[pl.ds(off, _CHUNK)]], pb, s_pg)
            gathers[b] = (g_t, g_p)

        for ci in range(min(_DEPTH - 1, n_chunks)):
            start_gather(ci, ci % _DEPTH)
        for ci in range(n_chunks):
            cur = ci % _DEPTH
            if ci + _DEPTH - 1 < n_chunks:
                start_gather(ci + _DEPTH - 1, (ci + _DEPTH - 1) % _DEPTH)
            g_t, g_p = gathers[cur]
            g_t.wait()
            g_p.wait()
            tb, pb, _, _, s_st = bufs[cur]

            def add_body(j, _, tb=tb, pb=pb):
                sl = pl.ds(j * _LANES, _LANES)
                for r in range(_CHUNK):
                    plsc.addupdate(tb.at[r, sl], pb[r, sl])
                return 0

            lax.fori_loop(0, n_vec, add_body, 0)
            stores[cur] = pltpu.async_copy(
                tb, out_hbm.at[pl.ds(base + ci * _CHUNK, _CHUNK)], s_st)
        for b in range(_DEPTH):
            if stores[b] is not None:
                stores[b].wait()

    return sc_embed


def kernel(token_ids, position_ids, token_table, pos_table):
    b, s = token_ids.shape
    _, hidden = token_table.shape
    n_rows = b * s
    tids = token_ids.reshape(n_rows).astype(jnp.int32)
    pids = position_ids.reshape(n_rows).astype(jnp.int32)
    out = _sc_embed_call(n_rows, hidden)(tids, pids, token_table, pos_table)
    return out.reshape(b, s, hidden)


# speedup vs baseline: 1.2573x; 1.2573x over previous
import functools

import jax
import jax.numpy as jnp
from jax import lax
from jax.experimental import pallas as pl
from jax.experimental.pallas import tpu as pltpu
from jax.experimental.pallas import tpu_sc as plsc

_LANES = 16
_NUM_WORKERS = 32
_CHUNK = 16
_DEPTH = 3


def _sc_embed_call(n_rows, hidden):
    per_w = n_rows // _NUM_WORKERS
    n_chunks = per_w // _CHUNK
    mesh = plsc.VectorSubcoreMesh(core_axis_name="c", subcore_axis_name="s")

    row_bufs = [pltpu.VMEM((_CHUNK, hidden), jnp.float32)
                for _ in range(2 * _DEPTH)]
    sems = [pltpu.SemaphoreType.DMA for _ in range(3 * _DEPTH)]

    @functools.partial(
        pl.kernel,
        mesh=mesh,
        out_type=jax.ShapeDtypeStruct((n_rows, hidden), jnp.float32),
        scratch_types=[
            pltpu.VMEM((per_w,), jnp.int32),
            pltpu.VMEM((per_w,), jnp.int32),
        ] + row_bufs + sems,
    )
    def sc_embed(tok_hbm, pos_hbm, ttab_hbm, ptab_hbm, out_hbm,
                 tidx_v, pidx_v, *scratch):
        wid = lax.axis_index("s") * 2 + lax.axis_index("c")
        base = wid * per_w
        pltpu.sync_copy(tok_hbm.at[pl.ds(base, per_w)], tidx_v)
        pltpu.sync_copy(pos_hbm.at[pl.ds(base, per_w)], pidx_v)

        n_vec = hidden // _LANES
        bufs = [(scratch[2 * b], scratch[2 * b + 1],
                 scratch[2 * _DEPTH + 3 * b],
                 scratch[2 * _DEPTH + 3 * b + 1],
                 scratch[2 * _DEPTH + 3 * b + 2]) for b in range(_DEPTH)]
        gathers = [None] * _DEPTH
        stores = [None] * _DEPTH

        def start_gather(ci, b):
            off = ci * _CHUNK
            tb, pb, s_tg, s_pg, _ = bufs[b]
            if stores[b] is not None:
                stores[b].wait()
                stores[b] = None
            g_t = pltpu.async_copy(
                ttab_hbm.at[tidx_v.at[pl.ds(off, _CHUNK)]], tb, s_tg)
            g_p = pltpu.async_copy(
                ptab_hbm.at[pidx_v.at[pl.ds(off, _CHUNK)]], pb, s_pg)
            gathers[b] = (g_t, g_p)

        for ci in range(min(_DEPTH - 1, n_chunks)):
            start_gather(ci, ci % _DEPTH)
        for ci in range(n_chunks):
            cur = ci % _DEPTH
            if ci + _DEPTH - 1 < n_chunks:
                start_gather(ci + _DEPTH - 1, (ci + _DEPTH - 1) % _DEPTH)
            g_t, g_p = gathers[cur]
            g_t.wait()
            g_p.wait()
            tb, pb, _, _, s_st = bufs[cur]

            @plsc.parallel_loop(0, n_vec, 1, unroll=2)
            def add_body(j, tb=tb, pb=pb):
                sl = pl.ds(j * _LANES, _LANES)
                for r in range(_CHUNK):
                    plsc.addupdate(tb.at[r, sl], pb[r, sl])
            stores[cur] = pltpu.async_copy(
                tb, out_hbm.at[pl.ds(base + ci * _CHUNK, _CHUNK)], s_st)
        for b in range(_DEPTH):
            if stores[b] is not None:
                stores[b].wait()

    return sc_embed


def kernel(token_ids, position_ids, token_table, pos_table):
    b, s = token_ids.shape
    _, hidden = token_table.shape
    n_rows = b * s
    tids = token_ids.reshape(n_rows).astype(jnp.int32)
    pids = position_ids.reshape(n_rows).astype(jnp.int32)
    out = _sc_embed_call(n_rows, hidden)(tids, pids, token_table, pos_table)
    return out.reshape(b, s, hidden)
